# Initial kernel scaffold; baseline (speedup 1.0000x reference)
import jax
import jax.numpy as jnp
from jax.experimental import pallas as pl


def kernel(event_types, numerical_features, event_table, W_num, b_num, W_out, b_out, gamma, beta):
    raise NotImplementedError("write your pallas kernel here")


# speedup vs baseline: 3.7323x; 3.7323x over previous
import functools

import jax
import jax.numpy as jnp
from jax import lax
from jax.experimental import pallas as pl
from jax.experimental.pallas import tpu as pltpu
from jax.experimental.pallas import tpu_sc as plsc

D_MODEL = 128
HALF = 64
N_NUM = 8

NC = 2
NS = 16
NW = NC * NS

GATHER_CHUNK = 512


def _proj_body(t_ref, wo_ref, p_ref):
    p_ref[...] = jnp.dot(t_ref[...], wo_ref[...][:HALF],
                         preferred_element_type=jnp.float32,
                         precision=lax.Precision.HIGHEST)


def _project_table(table, W_out, blk=2000):
    v = table.shape[0]
    return pl.pallas_call(
        _proj_body,
        grid=(v // blk,),
        in_specs=[
            pl.BlockSpec((blk, HALF), lambda i: (i, 0)),
            pl.BlockSpec((D_MODEL, D_MODEL), lambda i: (0, 0)),
        ],
        out_specs=pl.BlockSpec((blk, D_MODEL), lambda i: (i, 0)),
        out_shape=jax.ShapeDtypeStruct((v, D_MODEL), jnp.float32),
    )(table, W_out)


def _sc_gather_fn(n_tokens):
    b_per_w = n_tokens // NW
    n_chunks = b_per_w // GATHER_CHUNK

    mesh = plsc.VectorSubcoreMesh(core_axis_name="c", subcore_axis_name="s")

    @functools.partial(
        pl.kernel,
        mesh=mesh,
        out_type=jax.ShapeDtypeStruct((n_tokens, D_MODEL), jnp.float32),
        scratch_types=[
            pltpu.VMEM((GATHER_CHUNK,), jnp.int32),
            pltpu.VMEM((GATHER_CHUNK, D_MODEL), jnp.float32),
            pltpu.SemaphoreType.DMA,
        ],
    )
    def gather_k(table_hbm, idx_hbm, out_hbm, idx_v, rows_v, sem):
        wid = lax.axis_index("s") * NC + lax.axis_index("c")
        base = wid * b_per_w

        def body(i, carry):
            off = pl.multiple_of(base + i * GATHER_CHUNK, GATHER_CHUNK)
            pltpu.sync_copy(idx_hbm.at[pl.ds(off, GATHER_CHUNK)], idx_v)
            pltpu.async_copy(table_hbm.at[idx_v], rows_v, sem).wait()
            pltpu.sync_copy(rows_v, out_hbm.at[pl.ds(off, GATHER_CHUNK)])
            return carry

        lax.fori_loop(0, n_chunks, body, 0, unroll=False)

    return gather_k


def _tail_body(g_ref, nf_ref, wn_ref, bn_ref, wo_ref, bo_ref, gm_ref,
               bt_ref, o_ref):
    wo_b = wo_ref[...][HALF:]
    wc = jnp.dot(wn_ref[...], wo_b, preferred_element_type=jnp.float32,
                 precision=lax.Precision.HIGHEST)
    bc = jnp.dot(bn_ref[...], wo_b, preferred_element_type=jnp.float32,
                 precision=lax.Precision.HIGHEST) + bo_ref[...]
    contrib = jnp.dot(nf_ref[...], wc, preferred_element_type=jnp.float32,
                      precision=lax.Precision.HIGHEST)
    out = g_ref[...] + contrib + bc
    mean = jnp.mean(out, axis=-1, keepdims=True)
    cent = out - mean
    var = jnp.mean(cent * cent, axis=-1, keepdims=True)
    xhat = cent * lax.rsqrt(var + 1e-5)
    o_ref[...] = xhat * gm_ref[...] + bt_ref[...]


def _tc_tail(gathered, nf, W_num, b_num, W_out, b_out, gamma, beta,
             tok_blk=4096):
    n_tokens = gathered.shape[0]
    return pl.pallas_call(
        _tail_body,
        grid=(n_tokens // tok_blk,),
        in_specs=[
            pl.BlockSpec((tok_blk, D_MODEL), lambda i: (i, 0)),
            pl.BlockSpec((tok_blk, N_NUM), lambda i: (i, 0)),
            pl.BlockSpec((N_NUM, HALF), lambda i: (0, 0)),
            pl.BlockSpec((1, HALF), lambda i: (0, 0)),
            pl.BlockSpec((D_MODEL, D_MODEL), lambda i: (0, 0)),
            pl.BlockSpec((1, D_MODEL), lambda i: (0, 0)),
            pl.BlockSpec((1, D_MODEL), lambda i: (0, 0)),
            pl.BlockSpec((1, D_MODEL), lambda i: (0, 0)),
        ],
        out_specs=pl.BlockSpec((tok_blk, D_MODEL), lambda i: (i, 0)),
        out_shape=jax.ShapeDtypeStruct((n_tokens, D_MODEL), jnp.float32),
    )(gathered, nf, W_num, b_num, W_out, b_out, gamma, beta)


def kernel(event_types, numerical_features, event_table, W_num, b_num,
           W_out, b_out, gamma, beta):
    B, L = event_types.shape
    n_tokens = B * L
    idx = event_types.reshape(n_tokens).astype(jnp.int32)
    proj = _project_table(event_table, W_out)
    gathered = _sc_gather_fn(n_tokens)(proj, idx)
    nf = numerical_features.reshape(n_tokens, N_NUM)
    out = _tc_tail(gathered, nf, W_num, b_num.reshape(1, HALF), W_out,
                   b_out.reshape(1, D_MODEL), gamma.reshape(1, D_MODEL),
                   beta.reshape(1, D_MODEL))
    return out.reshape(B, L, D_MODEL)
